# baseline (speedup 1.0000x reference)
import functools

import jax
import jax.numpy as jnp
from jax import lax
from jax.experimental import pallas as pl
from jax.experimental.pallas import tpu as pltpu
from jax.experimental.pallas import tpu_sc as plsc

N = 10000
D = 128
NC = 2
NS = 16
L = 16
NW = NC * NS

CH = 64
CHUNKS = 160
EPW = CHUNKS * CH
E_PAD = NW * EPW
N_PAD = 10240
RPT = N_PAD // NS
RB = 4
RP = 8
K = 8
GROUPS = CHUNKS // K


def _sc_segment_sum(vecs, pk, zeros):
  mesh = plsc.VectorSubcoreMesh(
      core_axis_name="c", subcore_axis_name="s",
      num_cores=NC, num_subcores=NS)

  def body(vecs_h, pk_h, zeros_h, out_h,
           acc, rows_in, pkv, gsem, ssem, psem):
    c = lax.axis_index("c")
    s = lax.axis_index("s")
    wid = s * NC + c

    def wrap(x):
      return jnp.where(x >= CHUNKS, x - CHUNKS, x)

    def fire_pk(ci, slot):
      pltpu.async_copy(pk_h.at[:, wid, ci], pkv[slot], psem[slot])

    def wait_pk(slot):
      pltpu.make_async_copy(pk_h.at[:, 0, 0], pkv[slot], psem[slot]).wait()

    def fire_gather(pslot, gslot):
      pltpu.async_copy(vecs_h.at[pkv[pslot].at[0]], rows_in[gslot],
                       gsem[gslot])

    def wait_gather(gslot):
      pltpu.make_async_copy(
          vecs_h.at[pl.ds(0, CH)], rows_in[gslot], gsem[gslot]).wait()

    def fire_scatter(pslot, bslot, sslot):
      pltpu.async_copy(rows_in[bslot], acc.at[pkv[pslot].at[1]], ssem[sslot],
                       add=True)

    def wait_scatter(sslot):
      pltpu.make_async_copy(
          zeros_h.at[pl.ds(0, CH)], rows_in[sslot % RB], ssem[sslot]).wait()

    for i in range(RPT // 128):
      pltpu.sync_copy(zeros_h, acc.at[pl.ds(s * RPT + i * 128, 128)])
    plsc.subcore_barrier()

    for j in range(4):
      fire_pk(j, j)
    wait_pk(0)
    for j in range(2):
      pltpu.sync_copy(zeros_h.at[pl.ds(0, CH)], rows_in[2 + j])
      pltpu.async_copy(rows_in[2 + j], acc.at[pkv[0].at[1]], ssem[2 + j],
                       add=True)
    fire_gather(0, 0)
    wait_pk(1)
    fire_gather(1, 1)

    def group_body(g, carry):
      base = g * K
      for k in range(K):
        ci = base + k
        wait_gather(k % RB)
        wait_scatter((k + 2) % 4)

        def row_body(rb, carry2, _b=k % RB, _p=k % RP):
          a16 = pkv[_p][2, pl.ds(rb * L, L)]
          for i in range(L):
            a = lax.bitcast_convert_type(a16[i], jnp.float32)
            r = rb * L + i
            for gg in range(D // L):
              sl = pl.ds(gg * L, L)
              rows_in[_b][r, sl] = rows_in[_b][r, sl] * a
          return carry2

        lax.fori_loop(0, CH // L, row_body, 0)
        fire_scatter(k % RP, k % RB, k % 4)
        fire_pk(wrap(ci + 4), (k + 4) % RP)
        wait_pk((k + 2) % RP)
        fire_gather((k + 2) % RP, (k + 2) % RB)
      return carry

    lax.fori_loop(0, GROUPS, group_body, 0)

    wait_scatter(2)
    wait_scatter(3)
    for j in range(2):
      wait_gather(j % RB)
      wait_pk((2 + j) % RP)
    plsc.subcore_barrier()

    for i in range(RPT // 128):
      off = s * RPT + i * 128
      pltpu.sync_copy(acc.at[pl.ds(off, 128)], out_h.at[c, pl.ds(off, 128)])

  fn = pl.kernel(
      body,
      out_type=jax.ShapeDtypeStruct((NC, N_PAD, D), jnp.float32),
      mesh=mesh,
      compiler_params=pltpu.CompilerParams(needs_layout_passes=False),
      scratch_types=[
          pltpu.VMEM_SHARED((N_PAD, D), jnp.float32),
          [pltpu.VMEM((CH, D), jnp.float32) for _ in range(RB)],
          [pltpu.VMEM((3, CH), jnp.int32) for _ in range(RP)],
          [pltpu.SemaphoreType.DMA for _ in range(RB)],
          [pltpu.SemaphoreType.DMA for _ in range(4)],
          [pltpu.SemaphoreType.DMA for _ in range(RP)],
      ],
  )
  return fn(vecs, pk, zeros)


def _tc_self(vecs, self_weights):
  BR = 1000

  def body(v_ref, ws_ref, o_ref):
    o_ref[...] = jnp.dot(v_ref[...], ws_ref[...],
                         preferred_element_type=jnp.float32)

  return pl.pallas_call(
      body,
      grid=(N // BR,),
      in_specs=[
          pl.BlockSpec((BR, D), lambda i: (i, 0)),
          pl.BlockSpec((D, D), lambda i: (0, 0)),
      ],
      out_specs=pl.BlockSpec((BR, D), lambda i: (i, 0)),
      out_shape=jax.ShapeDtypeStruct((N, D), jnp.float32),
  )(vecs, self_weights)


def _tc_final(fs, partials, neigh_weights, offset, scale):
  BR = 1000

  def body(fs_ref, p0_ref, p1_ref, wn_ref, off_ref, sc_ref, o_ref):
    nm = p0_ref[0] + p1_ref[0]
    fn = jnp.dot(nm, wn_ref[...], preferred_element_type=jnp.float32)
    out = jnp.concatenate([fs_ref[...], fn], axis=1)
    mean = jnp.mean(out, axis=1, keepdims=True)
    var = jnp.mean(jnp.square(out - mean), axis=1, keepdims=True)
    out = (out - mean) / jnp.sqrt(var + 1e-9) * sc_ref[...] + off_ref[...]
    o_ref[...] = jnp.maximum(out, 0.0)

  return pl.pallas_call(
      body,
      grid=(N // BR,),
      in_specs=[
          pl.BlockSpec((BR, D), lambda i: (i, 0)),
          pl.BlockSpec((1, BR, D), lambda i: (0, i, 0)),
          pl.BlockSpec((1, BR, D), lambda i: (1, i, 0)),
          pl.BlockSpec((D, D), lambda i: (0, 0)),
          pl.BlockSpec((1, 2 * D), lambda i: (0, 0)),
          pl.BlockSpec((1, 2 * D), lambda i: (0, 0)),
      ],
      out_specs=pl.BlockSpec((BR, 2 * D), lambda i: (i, 0)),
      out_shape=jax.ShapeDtypeStruct((N, 2 * D), jnp.float32),
  )(fs, partials, partials, neigh_weights, offset, scale)


def kernel(vecs, edge_index, adj_values, nnz, len_feat,
           neigh_weights, self_weights, offset, scale):
  del nnz, len_feat
  E = edge_index.shape[1]
  pad = E_PAD - E
  pad_idx = jnp.mod(jnp.arange(pad, dtype=jnp.int32), N)
  src = jnp.concatenate([edge_index[0], pad_idx])
  dst = jnp.concatenate([edge_index[1], pad_idx])
  adj = jnp.concatenate([adj_values, jnp.zeros((pad,), jnp.float32)])
  adj_bits = lax.bitcast_convert_type(adj, jnp.int32)
  pk = jnp.concatenate([src, dst, adj_bits]).reshape(3, NW, CHUNKS, CH)
  zeros = jnp.zeros((128, D), jnp.float32)

  fs = _tc_self(vecs, self_weights)
  partials = _sc_segment_sum(vecs, pk, zeros)
  return _tc_final(fs, partials, neigh_weights, offset, scale)


# speedup vs baseline: 1.0382x; 1.0382x over previous
import functools

import jax
import jax.numpy as jnp
from jax import lax
from jax.experimental import pallas as pl
from jax.experimental.pallas import tpu as pltpu
from jax.experimental.pallas import tpu_sc as plsc

N = 10000
D = 128
NC = 2
NS = 16
L = 16
NW = NC * NS

CH = 64
CHUNKS = 160
EPW = CHUNKS * CH
E_PAD = NW * EPW
N_PAD = 10240
RPT = N_PAD // NS
RB = 4
RP = 8
K = 8
GROUPS = CHUNKS // K


def _sc_segment_sum(vecs, pk, zeros):
  mesh = plsc.VectorSubcoreMesh(
      core_axis_name="c", subcore_axis_name="s",
      num_cores=NC, num_subcores=NS)

  def body(vecs_h, pk_h, zeros_h, out_h,
           acc, rows_in, pkv, gsem, ssem, psem):
    c = lax.axis_index("c")
    s = lax.axis_index("s")
    wid = s * NC + c

    def wrap(x):
      return jnp.where(x >= CHUNKS, x - CHUNKS, x)

    def fire_pk(ci, slot):
      pltpu.async_copy(pk_h.at[wid, ci], pkv[slot], psem[slot])

    def wait_pk(slot):
      pltpu.make_async_copy(pk_h.at[0, 0], pkv[slot], psem[slot]).wait()

    def fire_gather(pslot, gslot):
      pltpu.async_copy(vecs_h.at[pkv[pslot].at[0]], rows_in[gslot],
                       gsem[gslot])

    def wait_gather(gslot):
      pltpu.make_async_copy(
          vecs_h.at[pl.ds(0, CH)], rows_in[gslot], gsem[gslot]).wait()

    def fire_scatter(pslot, bslot, sslot):
      pltpu.async_copy(rows_in[bslot], acc.at[pkv[pslot].at[1]], ssem[sslot],
                       add=True)

    def wait_scatter(sslot):
      pltpu.make_async_copy(
          zeros_h.at[pl.ds(0, CH)], rows_in[sslot % RB], ssem[sslot]).wait()

    for i in range(RPT // 128):
      pltpu.sync_copy(zeros_h, acc.at[pl.ds(s * RPT + i * 128, 128)])
    plsc.subcore_barrier()

    for j in range(4):
      fire_pk(j, j)
    wait_pk(0)
    for j in range(2):
      pltpu.sync_copy(zeros_h.at[pl.ds(0, CH)], rows_in[2 + j])
      pltpu.async_copy(rows_in[2 + j], acc.at[pkv[0].at[1]], ssem[2 + j],
                       add=True)
    fire_gather(0, 0)
    wait_pk(1)
    fire_gather(1, 1)

    def group_body(g, carry):
      base = g * K
      for k in range(K):
        ci = base + k
        wait_gather(k % RB)
        wait_scatter((k + 2) % 4)

        def row_body(rb, carry2, _b=k % RB, _p=k % RP):
          a16 = pkv[_p][2, pl.ds(rb * L, L)]
          for i in range(L):
            a = lax.bitcast_convert_type(a16[i], jnp.float32)
            r = rb * L + i
            for gg in range(D // L):
              sl = pl.ds(gg * L, L)
              rows_in[_b][r, sl] = rows_in[_b][r, sl] * a
          return carry2

        lax.fori_loop(0, CH // L, row_body, 0)
        fire_scatter(k % RP, k % RB, k % 4)
        fire_pk(wrap(ci + 4), (k + 4) % RP)
        wait_pk((k + 2) % RP)
        fire_gather((k + 2) % RP, (k + 2) % RB)
      return carry

    lax.fori_loop(0, GROUPS, group_body, 0)

    wait_scatter(2)
    wait_scatter(3)
    for j in range(2):
      wait_gather(j % RB)
      wait_pk((2 + j) % RP)
    plsc.subcore_barrier()

    for i in range(RPT // 128):
      off = s * RPT + i * 128
      pltpu.sync_copy(acc.at[pl.ds(off, 128)], out_h.at[c, pl.ds(off, 128)])

  fn = pl.kernel(
      body,
      out_type=jax.ShapeDtypeStruct((NC, N_PAD, D), jnp.float32),
      mesh=mesh,
      compiler_params=pltpu.CompilerParams(needs_layout_passes=False),
      scratch_types=[
          pltpu.VMEM_SHARED((N_PAD, D), jnp.float32),
          [pltpu.VMEM((CH, D), jnp.float32) for _ in range(RB)],
          [pltpu.VMEM((3, CH), jnp.int32) for _ in range(RP)],
          [pltpu.SemaphoreType.DMA for _ in range(RB)],
          [pltpu.SemaphoreType.DMA for _ in range(4)],
          [pltpu.SemaphoreType.DMA for _ in range(RP)],
      ],
  )
  return fn(vecs, pk, zeros)


def _tc_self(vecs, self_weights):
  BR = 1000

  def body(v_ref, ws_ref, o_ref):
    o_ref[...] = jnp.dot(v_ref[...], ws_ref[...],
                         preferred_element_type=jnp.float32)

  return pl.pallas_call(
      body,
      grid=(N // BR,),
      in_specs=[
          pl.BlockSpec((BR, D), lambda i: (i, 0)),
          pl.BlockSpec((D, D), lambda i: (0, 0)),
      ],
      out_specs=pl.BlockSpec((BR, D), lambda i: (i, 0)),
      out_shape=jax.ShapeDtypeStruct((N, D), jnp.float32),
  )(vecs, self_weights)


def _tc_final(fs, partials, neigh_weights, offset, scale):
  BR = 1000

  def body(fs_ref, p0_ref, p1_ref, wn_ref, off_ref, sc_ref, o_ref):
    nm = p0_ref[0] + p1_ref[0]
    fn = jnp.dot(nm, wn_ref[...], preferred_element_type=jnp.float32)
    out = jnp.concatenate([fs_ref[...], fn], axis=1)
    mean = jnp.mean(out, axis=1, keepdims=True)
    var = jnp.mean(jnp.square(out - mean), axis=1, keepdims=True)
    out = (out - mean) / jnp.sqrt(var + 1e-9) * sc_ref[...] + off_ref[...]
    o_ref[...] = jnp.maximum(out, 0.0)

  return pl.pallas_call(
      body,
      grid=(N // BR,),
      in_specs=[
          pl.BlockSpec((BR, D), lambda i: (i, 0)),
          pl.BlockSpec((1, BR, D), lambda i: (0, i, 0)),
          pl.BlockSpec((1, BR, D), lambda i: (1, i, 0)),
          pl.BlockSpec((D, D), lambda i: (0, 0)),
          pl.BlockSpec((1, 2 * D), lambda i: (0, 0)),
          pl.BlockSpec((1, 2 * D), lambda i: (0, 0)),
      ],
      out_specs=pl.BlockSpec((BR, 2 * D), lambda i: (i, 0)),
      out_shape=jax.ShapeDtypeStruct((N, 2 * D), jnp.float32),
  )(fs, partials, partials, neigh_weights, offset, scale)


def kernel(vecs, edge_index, adj_values, nnz, len_feat,
           neigh_weights, self_weights, offset, scale):
  del nnz, len_feat
  E = edge_index.shape[1]
  pad = E_PAD - E
  pad_idx = jnp.mod(jnp.arange(pad, dtype=jnp.int32), N)
  src = jnp.concatenate([edge_index[0], pad_idx])
  dst = jnp.concatenate([edge_index[1], pad_idx])
  adj = jnp.concatenate([adj_values, jnp.zeros((pad,), jnp.float32)])
  adj_bits = lax.bitcast_convert_type(adj, jnp.int32)
  pk = jnp.stack([src.reshape(NW, CHUNKS, CH),
                  dst.reshape(NW, CHUNKS, CH),
                  adj_bits.reshape(NW, CHUNKS, CH)], axis=2)
  zeros = jnp.zeros((128, D), jnp.float32)

  fs = _tc_self(vecs, self_weights)
  partials = _sc_segment_sum(vecs, pk, zeros)
  return _tc_final(fs, partials, neigh_weights, offset, scale)


# speedup vs baseline: 1.0606x; 1.0217x over previous
import functools

import jax
import jax.numpy as jnp
from jax import lax
from jax.experimental import pallas as pl
from jax.experimental.pallas import tpu as pltpu
from jax.experimental.pallas import tpu_sc as plsc

N = 10000
D = 128
NC = 2
NS = 16
L = 16
NW = NC * NS

CH = 64
CHUNKS = 160
EPW = CHUNKS * CH
E_PAD = NW * EPW
N_PAD = 10240
RPT = N_PAD // NS
RB = 4
RP = 8
K = 8
GROUPS = CHUNKS // K


def _sc_segment_sum(vecs, pk, zeros):
  mesh = plsc.VectorSubcoreMesh(
      core_axis_name="c", subcore_axis_name="s",
      num_cores=NC, num_subcores=NS)

  def body(vecs_h, pk_h, zeros_h, out_h,
           acc, rows_in, pkv, gsem, ssem, psem):
    c = lax.axis_index("c")
    s = lax.axis_index("s")
    wid = s * NC + c

    def wrap(x):
      return jnp.where(x >= CHUNKS, x - CHUNKS, x)

    def fire_pk(ci, slot):
      pltpu.async_copy(pk_h.at[wid, ci], pkv[slot], psem[slot])

    def wait_pk(slot):
      pltpu.make_async_copy(pk_h.at[0, 0], pkv[slot], psem[slot]).wait()

    def fire_gather(pslot, gslot):
      pltpu.async_copy(vecs_h.at[pkv[pslot].at[0]], rows_in[gslot],
                       gsem[gslot])

    def wait_gather(gslot):
      pltpu.make_async_copy(
          vecs_h.at[pl.ds(0, CH)], rows_in[gslot], gsem[gslot]).wait()

    def fire_scatter(pslot, bslot, sslot):
      pltpu.async_copy(rows_in[bslot], acc.at[pkv[pslot].at[1]], ssem[sslot],
                       add=True)

    def wait_scatter(sslot):
      pltpu.make_async_copy(
          zeros_h.at[pl.ds(0, CH)], rows_in[sslot % RB], ssem[sslot]).wait()

    for i in range(RPT // 128):
      pltpu.sync_copy(zeros_h, acc.at[pl.ds(s * RPT + i * 128, 128)])
    plsc.subcore_barrier()

    for j in range(4):
      fire_pk(j, j)
    wait_pk(0)
    for j in range(2):
      pltpu.sync_copy(zeros_h.at[pl.ds(0, CH)], rows_in[2 + j])
      pltpu.async_copy(rows_in[2 + j], acc.at[pkv[0].at[1]], ssem[2 + j],
                       add=True)
    fire_gather(0, 0)
    wait_pk(1)
    fire_gather(1, 1)

    def group_body(g, carry):
      base = g * K
      for k in range(K):
        ci = base + k
        wait_gather(k % RB)
        wait_scatter((k + 2) % 4)

        def row_body(rb, carry2, _b=k % RB, _p=k % RP):
          a16 = pkv[_p][2, pl.ds(rb * L, L)]
          for i in range(L):
            a = lax.bitcast_convert_type(a16[i], jnp.float32)
            r = rb * L + i
            for gg in range(D // L):
              sl = pl.ds(gg * L, L)
              rows_in[_b][r, sl] = rows_in[_b][r, sl] * a
          return carry2

        lax.fori_loop(0, CH // L, row_body, 0)
        fire_scatter(k % RP, k % RB, k % 4)
        fire_pk(wrap(ci + 4), (k + 4) % RP)
        wait_pk((k + 2) % RP)
        fire_gather((k + 2) % RP, (k + 2) % RB)
      return carry

    lax.fori_loop(0, GROUPS, group_body, 0)

    wait_scatter(2)
    wait_scatter(3)
    for j in range(2):
      wait_gather(j % RB)
      wait_pk((2 + j) % RP)
    plsc.subcore_barrier()

    for i in range(RPT // 128):
      off = s * RPT + i * 128
      pltpu.sync_copy(acc.at[pl.ds(off, 128)], out_h.at[c, pl.ds(off, 128)])

  fn = pl.kernel(
      body,
      out_type=jax.ShapeDtypeStruct((NC, N_PAD, D), jnp.float32),
      mesh=mesh,
      compiler_params=pltpu.CompilerParams(needs_layout_passes=False),
      scratch_types=[
          pltpu.VMEM_SHARED((N_PAD, D), jnp.float32),
          [pltpu.VMEM((CH, D), jnp.float32) for _ in range(RB)],
          [pltpu.VMEM((3, CH), jnp.int32) for _ in range(RP)],
          [pltpu.SemaphoreType.DMA for _ in range(RB)],
          [pltpu.SemaphoreType.DMA for _ in range(4)],
          [pltpu.SemaphoreType.DMA for _ in range(RP)],
      ],
  )
  return fn(vecs, pk, zeros)


def _tc_dense(vecs, partials, neigh_weights_perm, self_weights, offset, scale):
  BR = 2000

  def body(v_ref, p0_ref, p1_ref, wn_ref, ws_ref, off_ref, sc_ref, o_ref):
    v = v_ref[...]
    nm = p0_ref[0] + p1_ref[0]
    fs = jnp.dot(v, ws_ref[...], preferred_element_type=jnp.float32)
    fn = jnp.dot(nm, wn_ref[...], preferred_element_type=jnp.float32)
    out = jnp.concatenate([fs, fn], axis=1)
    mean = jnp.mean(out, axis=1, keepdims=True)
    var = jnp.mean(jnp.square(out - mean), axis=1, keepdims=True)
    out = (out - mean) / jnp.sqrt(var + 1e-9) * sc_ref[...] + off_ref[...]
    o_ref[...] = jnp.maximum(out, 0.0)

  return pl.pallas_call(
      body,
      grid=(N // BR,),
      in_specs=[
          pl.BlockSpec((BR, D), lambda i: (i, 0)),
          pl.BlockSpec((1, BR, D), lambda i: (0, i, 0)),
          pl.BlockSpec((1, BR, D), lambda i: (1, i, 0)),
          pl.BlockSpec((D, D), lambda i: (0, 0)),
          pl.BlockSpec((D, D), lambda i: (0, 0)),
          pl.BlockSpec((1, 2 * D), lambda i: (0, 0)),
          pl.BlockSpec((1, 2 * D), lambda i: (0, 0)),
      ],
      out_specs=pl.BlockSpec((BR, 2 * D), lambda i: (i, 0)),
      out_shape=jax.ShapeDtypeStruct((N, 2 * D), jnp.float32),
  )(vecs, partials, partials, neigh_weights_perm, self_weights, offset, scale)


def kernel(vecs, edge_index, adj_values, nnz, len_feat,
           neigh_weights, self_weights, offset, scale):
  del nnz, len_feat
  E = edge_index.shape[1]
  pad = E_PAD - E
  pad_idx = jnp.mod(jnp.arange(pad, dtype=jnp.int32), N)
  src = jnp.concatenate([edge_index[0], pad_idx])
  dst = jnp.concatenate([edge_index[1], pad_idx])
  adj = jnp.concatenate([adj_values, jnp.zeros((pad,), jnp.float32)])
  adj_bits = lax.bitcast_convert_type(adj, jnp.int32)
  pk = jnp.stack([src.reshape(NW, CHUNKS, CH),
                  dst.reshape(NW, CHUNKS, CH),
                  adj_bits.reshape(NW, CHUNKS, CH)], axis=2)
  zeros = jnp.zeros((128, D), jnp.float32)

  partials = _sc_segment_sum(vecs, pk, zeros)
  return _tc_dense(vecs, partials, neigh_weights, self_weights, offset, scale)
